# baseline (speedup 1.0000x reference)
import functools

import jax
import jax.numpy as jnp
from jax import lax
from jax.experimental import pallas as pl
from jax.experimental.pallas import tpu as pltpu
from jax.experimental.pallas import tpu_sc as plsc

N = 10000
E = 320000
D = 128
C = 10
G = 64

NC = 2
NS = 16
LANES = 16
EPC = E // NC
EPT = EPC // NS
CHUNK = 80
CPT = EPT // CHUNK
K2 = 2
NPAIR = CPT // (2 * K2)
N_PAD = 10240
RPT = N_PAD // NS

BLK = 1000
GRID = N // BLK

_mesh = plsc.VectorSubcoreMesh(core_axis_name="c", subcore_axis_name="s")
_sc_params = pltpu.CompilerParams(use_tc_tiling_on_sc=False)


def _sc_deg(dst2d, zeros16):
    @functools.partial(
        pl.kernel,
        mesh=_mesh,
        out_type=jax.ShapeDtypeStruct((NC * N_PAD, LANES), jnp.float32),
        scratch_types=[
            pltpu.VMEM((CPT, CHUNK), jnp.int32),
            pltpu.VMEM((CHUNK, LANES), jnp.float32),
            pltpu.VMEM_SHARED((N_PAD, LANES), jnp.float32),
            pltpu.SemaphoreType.DMA,
        ],
        compiler_params=_sc_params,
    )
    def k(dst_hbm, z_hbm, out_hbm, didx, ones_v, acc_sh, ssem):
        c = lax.axis_index("c")
        s = lax.axis_index("s")

        @pl.loop(0, CHUNK)
        def _(i):
            ones_v[i, :] = jnp.ones((LANES,), jnp.float32)

        pltpu.sync_copy(z_hbm.at[pl.ds(s * RPT, RPT)],
                        acc_sh.at[pl.ds(s * RPT, RPT)])

        cbase = (c * NS + s) * CPT
        pltpu.sync_copy(dst_hbm.at[pl.ds(cbase, CPT)], didx)

        plsc.subcore_barrier()

        @pl.loop(0, CPT)
        def _(t):
            pltpu.async_copy(ones_v, acc_sh.at[didx.at[t]], ssem, add=True)

        @pl.loop(0, CPT)
        def _(t):
            pltpu.make_async_copy(ones_v, acc_sh.at[didx.at[t]], ssem).wait()

        plsc.subcore_barrier()

        pltpu.sync_copy(acc_sh.at[pl.ds(s * RPT, RPT)],
                        out_hbm.at[pl.ds(c * N_PAD + s * RPT, RPT)])

    return k(dst2d, zeros16)


def _sc_prop(g, src2d, dst2d, zeros128):
    @functools.partial(
        pl.kernel,
        mesh=_mesh,
        out_type=jax.ShapeDtypeStruct((NC * N_PAD, D), jnp.float32),
        scratch_types=[
            pltpu.VMEM((2, K2, CHUNK), jnp.int32),
            pltpu.VMEM((2, K2, CHUNK), jnp.int32),
            pltpu.VMEM((2, K2, CHUNK, D), jnp.float32),
            pltpu.VMEM_SHARED((N_PAD, D), jnp.float32),
            pltpu.SemaphoreType.DMA,
            pltpu.SemaphoreType.DMA,
        ],
        compiler_params=_sc_params,
    )
    def k(g_hbm, src_hbm, dst_hbm, z_hbm, out_hbm, sidx, didx, rows, acc_sh,
          gsem, ssem):
        c = lax.axis_index("c")
        s = lax.axis_index("s")

        pltpu.sync_copy(z_hbm.at[pl.ds(s * RPT, RPT)],
                        acc_sh.at[pl.ds(s * RPT, RPT)])

        cbase = (c * NS + s) * CPT

        plsc.subcore_barrier()

        def idxload(grp, p):
            pltpu.sync_copy(src_hbm.at[pl.ds(cbase + grp * K2, K2)],
                            sidx.at[p])
            pltpu.sync_copy(dst_hbm.at[pl.ds(cbase + grp * K2, K2)],
                            didx.at[p])

        def gath_issue(p):
            for j in range(K2):
                pltpu.async_copy(g_hbm.at[sidx.at[p, j]], rows.at[p, j],
                                 gsem)

        def gath_wait(p):
            for j in range(K2):
                pltpu.make_async_copy(g_hbm.at[sidx.at[p, j]],
                                      rows.at[p, j], gsem).wait()

        def scat_issue(p):
            for j in range(K2):
                pltpu.async_copy(rows.at[p, j], acc_sh.at[didx.at[p, j]],
                                 ssem, add=True)

        def scat_wait(p):
            for j in range(K2):
                pltpu.make_async_copy(rows.at[p, j],
                                      acc_sh.at[didx.at[p, j]], ssem).wait()

        idxload(0, 0)
        gath_issue(0)

        @pl.loop(0, NPAIR)
        def _(i):
            gath_wait(0)
            scat_issue(0)

            @pl.when(i > 0)
            def _():
                scat_wait(1)

            idxload(2 * i + 1, 1)
            gath_issue(1)
            scat_wait(0)

            @pl.when(i + 1 < NPAIR)
            def _():
                idxload(2 * i + 2, 0)
                gath_issue(0)

            gath_wait(1)
            scat_issue(1)

        scat_wait(1)

        for t in range(2 * NPAIR * K2, CPT):
            pltpu.sync_copy(src_hbm.at[pl.ds(cbase + t, 1)],
                            sidx.at[0, pl.ds(0, 1)])
            pltpu.sync_copy(dst_hbm.at[pl.ds(cbase + t, 1)],
                            didx.at[0, pl.ds(0, 1)])
            pltpu.async_copy(g_hbm.at[sidx.at[0, 0]], rows.at[0, 0],
                             gsem).wait()
            pltpu.sync_copy(rows.at[0, 0], acc_sh.at[didx.at[0, 0]],
                            add=True)

        plsc.subcore_barrier()

        pltpu.sync_copy(acc_sh.at[pl.ds(s * RPT, RPT)],
                        out_hbm.at[pl.ds(c * N_PAD + s * RPT, RPT)])

    return k(g, src2d, dst2d, zeros128)


def _leaky(v):
    return jnp.where(v >= 0, v, 0.01 * v)


def _tc_matmul(x, w):
    def body(x_ref, w_ref, o_ref):
        o_ref[...] = jnp.dot(x_ref[...], w_ref[...],
                             preferred_element_type=jnp.float32)

    return pl.pallas_call(
        body,
        grid=(GRID,),
        in_specs=[
            pl.BlockSpec((BLK, D), lambda i: (i, 0)),
            pl.BlockSpec((D, D), lambda i: (0, 0)),
        ],
        out_specs=pl.BlockSpec((BLK, D), lambda i: (i, 0)),
        out_shape=jax.ShapeDtypeStruct((N, D), jnp.float32),
    )(x, w)


def _tc_prep(degp, t1):
    def body(p0_ref, p1_ref, t_ref, g_ref, dinv_ref):
        deg = p0_ref[0, :, :1] + p1_ref[0, :, :1] + 1.0
        dinv = lax.rsqrt(deg)
        dinv_ref[...] = dinv
        g_ref[...] = dinv * t_ref[...]

    return pl.pallas_call(
        body,
        grid=(GRID,),
        in_specs=[
            pl.BlockSpec((1, BLK, LANES), lambda i: (0, i, 0)),
            pl.BlockSpec((1, BLK, LANES), lambda i: (1, i, 0)),
            pl.BlockSpec((BLK, D), lambda i: (i, 0)),
        ],
        out_specs=[
            pl.BlockSpec((BLK, D), lambda i: (i, 0)),
            pl.BlockSpec((BLK, 1), lambda i: (i, 0)),
        ],
        out_shape=[
            jax.ShapeDtypeStruct((N, D), jnp.float32),
            jax.ShapeDtypeStruct((N, 1), jnp.float32),
        ],
    )(degp, degp, t1)


def _tc_mid(sp, g_prev, dinv, b, w_next):
    def body(s0_ref, s1_ref, g_ref, d_ref, b_ref, w_ref, o_ref):
        dinv = d_ref[...]
        h = _leaky(dinv * (s0_ref[0] + s1_ref[0] + g_ref[...])
                   + b_ref[...])
        o_ref[...] = dinv * jnp.dot(h, w_ref[...],
                                    preferred_element_type=jnp.float32)

    return pl.pallas_call(
        body,
        grid=(GRID,),
        in_specs=[
            pl.BlockSpec((1, BLK, D), lambda i: (0, i, 0)),
            pl.BlockSpec((1, BLK, D), lambda i: (1, i, 0)),
            pl.BlockSpec((BLK, D), lambda i: (i, 0)),
            pl.BlockSpec((BLK, 1), lambda i: (i, 0)),
            pl.BlockSpec((1, D), lambda i: (0, 0)),
            pl.BlockSpec((D, D), lambda i: (0, 0)),
        ],
        out_specs=pl.BlockSpec((BLK, D), lambda i: (i, 0)),
        out_shape=jax.ShapeDtypeStruct((N, D), jnp.float32),
    )(sp, sp, g_prev, dinv, b, w_next)


def _tc_final(sp, g_prev, dinv, b, batch3d, wc, bc):
    def body(s0_ref, s1_ref, g_ref, d_ref, b_ref, bat_ref, wc_ref, bc_ref,
             o_ref, sums_ref, cnt_ref):
        i = pl.program_id(0)

        @pl.when(i == 0)
        def _():
            sums_ref[...] = jnp.zeros_like(sums_ref)
            cnt_ref[...] = jnp.zeros_like(cnt_ref)

        dinv = d_ref[...]
        h = _leaky(dinv * (s0_ref[0] + s1_ref[0] + g_ref[...])
                   + b_ref[...])
        brow = bat_ref[0]
        gids = lax.broadcasted_iota(jnp.int32, (G, BLK), 0)
        mask = (brow == gids).astype(jnp.float32)
        sums_ref[...] += jnp.dot(mask, h, preferred_element_type=jnp.float32)
        cnt_ref[:, :1] += jnp.sum(mask, axis=1, keepdims=True)

        @pl.when(i == GRID - 1)
        def _():
            pooled = sums_ref[...] / jnp.maximum(cnt_ref[:, :1], 1.0)
            logits = jnp.dot(pooled, wc_ref[...],
                             preferred_element_type=jnp.float32) + bc_ref[...]
            m = jnp.max(logits, axis=1, keepdims=True)
            e = jnp.exp(logits - m)
            o_ref[...] = e / jnp.sum(e, axis=1, keepdims=True)

    return pl.pallas_call(
        body,
        grid=(GRID,),
        in_specs=[
            pl.BlockSpec((1, BLK, D), lambda i: (0, i, 0)),
            pl.BlockSpec((1, BLK, D), lambda i: (1, i, 0)),
            pl.BlockSpec((BLK, D), lambda i: (i, 0)),
            pl.BlockSpec((BLK, 1), lambda i: (i, 0)),
            pl.BlockSpec((1, D), lambda i: (0, 0)),
            pl.BlockSpec((1, 1, BLK), lambda i: (i, 0, 0)),
            pl.BlockSpec((D, C), lambda i: (0, 0)),
            pl.BlockSpec((1, C), lambda i: (0, 0)),
        ],
        out_specs=pl.BlockSpec((G, C), lambda i: (0, 0)),
        out_shape=jax.ShapeDtypeStruct((G, C), jnp.float32),
        scratch_shapes=[
            pltpu.VMEM((G, D), jnp.float32),
            pltpu.VMEM((G, D), jnp.float32),
        ],
    )(sp, sp, g_prev, dinv, b, batch3d, wc, bc)


def kernel(x, edge_index, batch, W1, b1, W2, b2, W3, b3, Wc, bc):
    src2d = edge_index[0].reshape(E // CHUNK, CHUNK)
    dst2d = edge_index[1].reshape(E // CHUNK, CHUNK)
    zeros16 = jnp.zeros((N_PAD, LANES), jnp.float32)
    zeros128 = jnp.zeros((N_PAD, D), jnp.float32)

    degp = _sc_deg(dst2d, zeros16).reshape(NC, N_PAD, LANES)
    t1 = _tc_matmul(x, W1)
    g1, dinv = _tc_prep(degp, t1)

    s1 = _sc_prop(g1, src2d, dst2d, zeros128).reshape(NC, N_PAD, D)
    g2 = _tc_mid(s1, g1, dinv, b1.reshape(1, D), W2)
    s2 = _sc_prop(g2, src2d, dst2d, zeros128).reshape(NC, N_PAD, D)
    g3 = _tc_mid(s2, g2, dinv, b2.reshape(1, D), W3)
    s3 = _sc_prop(g3, src2d, dst2d, zeros128).reshape(NC, N_PAD, D)

    return _tc_final(s3, g3, dinv, b3.reshape(1, D),
                     batch.reshape(GRID, 1, BLK), Wc, bc.reshape(1, C))


# speedup vs baseline: 1.0561x; 1.0561x over previous
import functools

import jax
import jax.numpy as jnp
from jax import lax
from jax.experimental import pallas as pl
from jax.experimental.pallas import tpu as pltpu
from jax.experimental.pallas import tpu_sc as plsc

N = 10000
E = 320000
D = 128
C = 10
G = 64

NC = 2
NS = 16
LANES = 16
EPC = E // NC
EPT = EPC // NS
CHUNK = 80
CPT = EPT // CHUNK
K2 = 2
NPAIR = CPT // (2 * K2)
N_PAD = 10240
RPT = N_PAD // NS

BLK = 1000
GRID = N // BLK

_mesh = plsc.VectorSubcoreMesh(core_axis_name="c", subcore_axis_name="s")
_sc_params = pltpu.CompilerParams(use_tc_tiling_on_sc=False)


def _sc_deg(dst2d, zeros16):
    @functools.partial(
        pl.kernel,
        mesh=_mesh,
        out_type=jax.ShapeDtypeStruct((NC * N_PAD, LANES), jnp.float32),
        scratch_types=[
            pltpu.VMEM((CPT, CHUNK), jnp.int32),
            pltpu.VMEM((CHUNK, LANES), jnp.float32),
            pltpu.VMEM_SHARED((N_PAD, LANES), jnp.float32),
            pltpu.SemaphoreType.DMA,
        ],
        compiler_params=_sc_params,
    )
    def k(dst_hbm, z_hbm, out_hbm, didx, ones_v, acc_sh, ssem):
        c = lax.axis_index("c")
        s = lax.axis_index("s")

        @pl.loop(0, CHUNK)
        def _(i):
            ones_v[i, :] = jnp.ones((LANES,), jnp.float32)

        pltpu.sync_copy(z_hbm.at[pl.ds(s * RPT, RPT)],
                        acc_sh.at[pl.ds(s * RPT, RPT)])

        cbase = (c * NS + s) * CPT
        pltpu.sync_copy(dst_hbm.at[pl.ds(cbase, CPT)], didx)

        plsc.subcore_barrier()

        @pl.loop(0, CPT)
        def _(t):
            pltpu.async_copy(ones_v, acc_sh.at[didx.at[t]], ssem, add=True)

        @pl.loop(0, CPT)
        def _(t):
            pltpu.make_async_copy(ones_v, acc_sh.at[didx.at[t]], ssem).wait()

        plsc.subcore_barrier()

        pltpu.sync_copy(acc_sh.at[pl.ds(s * RPT, RPT)],
                        out_hbm.at[pl.ds(c * N_PAD + s * RPT, RPT)])

    return k(dst2d, zeros16)


def _sc_prop(g, eidx2d, zeros128):
    @functools.partial(
        pl.kernel,
        mesh=_mesh,
        out_type=jax.ShapeDtypeStruct((NC * N_PAD, D), jnp.float32),
        scratch_types=[
            pltpu.VMEM((2, K2, 2, CHUNK), jnp.int32),
            pltpu.VMEM((2, K2, CHUNK, D), jnp.float32),
            pltpu.VMEM_SHARED((N_PAD, D), jnp.float32),
            pltpu.SemaphoreType.DMA,
            pltpu.SemaphoreType.DMA,
        ],
        compiler_params=_sc_params,
    )
    def k(g_hbm, eidx_hbm, z_hbm, out_hbm, bidx, rows, acc_sh,
          gsem, ssem):
        c = lax.axis_index("c")
        s = lax.axis_index("s")

        pltpu.sync_copy(z_hbm.at[pl.ds(s * RPT, RPT)],
                        acc_sh.at[pl.ds(s * RPT, RPT)])

        cbase = (c * NS + s) * CPT

        plsc.subcore_barrier()

        def idxload(grp, p):
            pltpu.sync_copy(eidx_hbm.at[pl.ds(cbase + grp * K2, K2)],
                            bidx.at[p])

        def gath_issue(p):
            for j in range(K2):
                pltpu.async_copy(g_hbm.at[bidx.at[p, j, 0]], rows.at[p, j],
                                 gsem)

        def gath_wait(p):
            for j in range(K2):
                pltpu.make_async_copy(g_hbm.at[bidx.at[p, j, 0]],
                                      rows.at[p, j], gsem).wait()

        def scat_issue(p):
            for j in range(K2):
                pltpu.async_copy(rows.at[p, j], acc_sh.at[bidx.at[p, j, 1]],
                                 ssem, add=True)

        def scat_wait(p):
            for j in range(K2):
                pltpu.make_async_copy(rows.at[p, j],
                                      acc_sh.at[bidx.at[p, j, 1]],
                                      ssem).wait()

        idxload(0, 0)
        gath_issue(0)

        @pl.loop(0, NPAIR)
        def _(i):
            gath_wait(0)
            scat_issue(0)

            @pl.when(i > 0)
            def _():
                scat_wait(1)

            idxload(2 * i + 1, 1)
            gath_issue(1)
            scat_wait(0)

            @pl.when(i + 1 < NPAIR)
            def _():
                idxload(2 * i + 2, 0)
                gath_issue(0)

            gath_wait(1)
            scat_issue(1)

        scat_wait(1)

        for t in range(2 * NPAIR * K2, CPT):
            pltpu.sync_copy(eidx_hbm.at[pl.ds(cbase + t, 1)],
                            bidx.at[0, pl.ds(0, 1)])
            pltpu.async_copy(g_hbm.at[bidx.at[0, 0, 0]], rows.at[0, 0],
                             gsem).wait()
            pltpu.sync_copy(rows.at[0, 0], acc_sh.at[bidx.at[0, 0, 1]],
                            add=True)

        plsc.subcore_barrier()

        pltpu.sync_copy(acc_sh.at[pl.ds(s * RPT, RPT)],
                        out_hbm.at[pl.ds(c * N_PAD + s * RPT, RPT)])

    return k(g, eidx2d, zeros128)


def _leaky(v):
    return jnp.where(v >= 0, v, 0.01 * v)


def _tc_matmul(x, w):
    def body(x_ref, w_ref, o_ref):
        o_ref[...] = jnp.dot(x_ref[...], w_ref[...],
                             preferred_element_type=jnp.float32)

    return pl.pallas_call(
        body,
        grid=(GRID,),
        in_specs=[
            pl.BlockSpec((BLK, D), lambda i: (i, 0)),
            pl.BlockSpec((D, D), lambda i: (0, 0)),
        ],
        out_specs=pl.BlockSpec((BLK, D), lambda i: (i, 0)),
        out_shape=jax.ShapeDtypeStruct((N, D), jnp.float32),
    )(x, w)


def _tc_prep(degp, t1):
    def body(p0_ref, p1_ref, t_ref, g_ref, dinv_ref):
        deg = p0_ref[0, :, :1] + p1_ref[0, :, :1] + 1.0
        dinv = lax.rsqrt(deg)
        dinv_ref[...] = dinv
        g_ref[...] = dinv * t_ref[...]

    return pl.pallas_call(
        body,
        grid=(GRID,),
        in_specs=[
            pl.BlockSpec((1, BLK, LANES), lambda i: (0, i, 0)),
            pl.BlockSpec((1, BLK, LANES), lambda i: (1, i, 0)),
            pl.BlockSpec((BLK, D), lambda i: (i, 0)),
        ],
        out_specs=[
            pl.BlockSpec((BLK, D), lambda i: (i, 0)),
            pl.BlockSpec((BLK, 1), lambda i: (i, 0)),
        ],
        out_shape=[
            jax.ShapeDtypeStruct((N, D), jnp.float32),
            jax.ShapeDtypeStruct((N, 1), jnp.float32),
        ],
    )(degp, degp, t1)


def _tc_mid(sp, g_prev, dinv, b, w_next):
    def body(s0_ref, s1_ref, g_ref, d_ref, b_ref, w_ref, o_ref):
        dinv = d_ref[...]
        h = _leaky(dinv * (s0_ref[0] + s1_ref[0] + g_ref[...])
                   + b_ref[...])
        o_ref[...] = dinv * jnp.dot(h, w_ref[...],
                                    preferred_element_type=jnp.float32)

    return pl.pallas_call(
        body,
        grid=(GRID,),
        in_specs=[
            pl.BlockSpec((1, BLK, D), lambda i: (0, i, 0)),
            pl.BlockSpec((1, BLK, D), lambda i: (1, i, 0)),
            pl.BlockSpec((BLK, D), lambda i: (i, 0)),
            pl.BlockSpec((BLK, 1), lambda i: (i, 0)),
            pl.BlockSpec((1, D), lambda i: (0, 0)),
            pl.BlockSpec((D, D), lambda i: (0, 0)),
        ],
        out_specs=pl.BlockSpec((BLK, D), lambda i: (i, 0)),
        out_shape=jax.ShapeDtypeStruct((N, D), jnp.float32),
    )(sp, sp, g_prev, dinv, b, w_next)


def _tc_final(sp, g_prev, dinv, b, batch3d, wc, bc):
    def body(s0_ref, s1_ref, g_ref, d_ref, b_ref, bat_ref, wc_ref, bc_ref,
             o_ref, sums_ref, cnt_ref):
        i = pl.program_id(0)

        @pl.when(i == 0)
        def _():
            sums_ref[...] = jnp.zeros_like(sums_ref)
            cnt_ref[...] = jnp.zeros_like(cnt_ref)

        dinv = d_ref[...]
        h = _leaky(dinv * (s0_ref[0] + s1_ref[0] + g_ref[...])
                   + b_ref[...])
        brow = bat_ref[0]
        gids = lax.broadcasted_iota(jnp.int32, (G, BLK), 0)
        mask = (brow == gids).astype(jnp.float32)
        sums_ref[...] += jnp.dot(mask, h, preferred_element_type=jnp.float32)
        cnt_ref[:, :1] += jnp.sum(mask, axis=1, keepdims=True)

        @pl.when(i == GRID - 1)
        def _():
            pooled = sums_ref[...] / jnp.maximum(cnt_ref[:, :1], 1.0)
            logits = jnp.dot(pooled, wc_ref[...],
                             preferred_element_type=jnp.float32) + bc_ref[...]
            m = jnp.max(logits, axis=1, keepdims=True)
            e = jnp.exp(logits - m)
            o_ref[...] = e / jnp.sum(e, axis=1, keepdims=True)

    return pl.pallas_call(
        body,
        grid=(GRID,),
        in_specs=[
            pl.BlockSpec((1, BLK, D), lambda i: (0, i, 0)),
            pl.BlockSpec((1, BLK, D), lambda i: (1, i, 0)),
            pl.BlockSpec((BLK, D), lambda i: (i, 0)),
            pl.BlockSpec((BLK, 1), lambda i: (i, 0)),
            pl.BlockSpec((1, D), lambda i: (0, 0)),
            pl.BlockSpec((1, 1, BLK), lambda i: (i, 0, 0)),
            pl.BlockSpec((D, C), lambda i: (0, 0)),
            pl.BlockSpec((1, C), lambda i: (0, 0)),
        ],
        out_specs=pl.BlockSpec((G, C), lambda i: (0, 0)),
        out_shape=jax.ShapeDtypeStruct((G, C), jnp.float32),
        scratch_shapes=[
            pltpu.VMEM((G, D), jnp.float32),
            pltpu.VMEM((G, D), jnp.float32),
        ],
    )(sp, sp, g_prev, dinv, b, batch3d, wc, bc)


def kernel(x, edge_index, batch, W1, b1, W2, b2, W3, b3, Wc, bc):
    dst2d = edge_index[1].reshape(E // CHUNK, CHUNK)
    eidx2d = jnp.stack([edge_index[0].reshape(E // CHUNK, CHUNK), dst2d],
                       axis=1)
    zeros16 = jnp.zeros((N_PAD, LANES), jnp.float32)
    zeros128 = jnp.zeros((N_PAD, D), jnp.float32)

    degp = _sc_deg(dst2d, zeros16).reshape(NC, N_PAD, LANES)
    t1 = _tc_matmul(x, W1)
    g1, dinv = _tc_prep(degp, t1)

    s1 = _sc_prop(g1, eidx2d, zeros128).reshape(NC, N_PAD, D)
    g2 = _tc_mid(s1, g1, dinv, b1.reshape(1, D), W2)
    s2 = _sc_prop(g2, eidx2d, zeros128).reshape(NC, N_PAD, D)
    g3 = _tc_mid(s2, g2, dinv, b2.reshape(1, D), W3)
    s3 = _sc_prop(g3, eidx2d, zeros128).reshape(NC, N_PAD, D)

    return _tc_final(s3, g3, dinv, b3.reshape(1, D),
                     batch.reshape(GRID, 1, BLK), Wc, bc.reshape(1, C))
